# baseline (speedup 1.0000x reference)
import functools

import jax
import jax.numpy as jnp
import numpy as np
from jax import lax
from jax.experimental import pallas as pl
from jax.experimental.pallas import tpu as pltpu
from jax.experimental.pallas import tpu_sc as plsc

N = 1024
C = 256
O = 256
K = 20
NC = 2
NS = 16
L = 16
NW = NC * NS
RPW = N // NW
NCHUNK = N // L
NEG_INF = np.float32(-np.inf)


def _dist_body(x_ref, o_ref):
    x = x_ref[...]
    g = lax.dot_general(x, x, (((0,), (0,)), ((), ())),
                        preferred_element_type=jnp.float32)
    x2 = jnp.sum(x * x, axis=0)
    o_ref[...] = 2.0 * g - x2[None, :] - x2[:, None]


def _dist(x):
    return pl.pallas_call(
        _dist_body,
        out_shape=jax.ShapeDtypeStruct((N, N), jnp.float32),
    )(x)


def _iota16():
    return lax.broadcasted_iota(jnp.int32, (L,), 0)


_GATHER_DNUMS = lax.GatherDimensionNumbers(
    offset_dims=(), collapsed_slice_dims=(0,), start_index_map=(0,))


def _shuffle(v, perm):
    return lax.gather(v, perm[:, None], _GATHER_DNUMS, (1,),
                      mode=lax.GatherScatterMode.PROMISE_IN_BOUNDS)


def _vmax16(v):
    for s in (8, 4, 2, 1):
        v = jnp.maximum(v, _shuffle(v, _iota16() ^ s))
    return v


def _vmin16(v):
    for s in (8, 4, 2, 1):
        v = jnp.minimum(v, _shuffle(v, _iota16() ^ s))
    return v


def _topk_into(row_v, topidx_v, iot, big):
    nolane = np.int32(99)

    def csum(c, carry):
        s0, s1 = carry
        b = c * 2 * L
        cmax = _vmax16(jnp.maximum(row_v[pl.ds(b, L)],
                                   row_v[pl.ds(b + L, L)]))
        cl = c & (L - 1)
        g = c >> 4
        s0 = jnp.where(iot == jnp.where(g == 0, cl, nolane), cmax, s0)
        s1 = jnp.where(iot == jnp.where(g == 1, cl, nolane), cmax, s1)
        return s0, s1
    init = row_v[pl.ds(0, L)]
    sm = lax.fori_loop(0, 2 * L, csum, (init, init))

    def extract(k, carry):
        s0, s1, idx_a, idx_b = carry
        gm = _vmax16(jnp.maximum(s0, s1))
        key = jnp.minimum(jnp.where(s0 == gm, iot, big),
                          jnp.where(s1 == gm, iot + L, big))
        cstar = _vmin16(key)
        cs = cstar[0]
        v0 = row_v[pl.ds(cs * 2 * L, L)]
        v1 = row_v[pl.ds(cs * 2 * L + L, L)]
        lane = _vmin16(jnp.minimum(jnp.where(v0 == gm, iot, big),
                                   jnp.where(v1 == gm, iot + L, big)))
        gi = cstar * 2 * L + lane
        v0 = jnp.where(iot == lane, NEG_INF, v0)
        v1 = jnp.where(iot == lane - L, NEG_INF, v1)
        row_v[pl.ds(cs * 2 * L, L)] = v0
        row_v[pl.ds(cs * 2 * L + L, L)] = v1
        nmax = _vmax16(jnp.maximum(v0, v1))
        cl = cs & (L - 1)
        g = cs >> 4
        nolane = np.int32(99)
        s0 = jnp.where(iot == jnp.where(g == 0, cl, nolane), nmax, s0)
        s1 = jnp.where(iot == jnp.where(g == 1, cl, nolane), nmax, s1)
        idx_a = jnp.where(iot == k, gi, idx_a)
        idx_b = jnp.where(iot == k - 8, gi, idx_b)
        return s0, s1, idx_a, idx_b
    carry = lax.fori_loop(0, K, extract, (sm[0], sm[1], iot, iot))
    topidx_v[pl.ds(0, L)] = carry[2]
    topidx_v[pl.ds(8, L)] = carry[3]


def _summary2(row_v, iot):
    sm = []
    for g in range(2):
        s_g = None
        for l in range(L):
            b = (g * L + l) * 2 * L
            cmax = _vmax16(jnp.maximum(row_v[pl.ds(b, L)],
                                       row_v[pl.ds(b + L, L)]))
            s_g = cmax if s_g is None else jnp.where(iot == l, cmax, s_g)
        sm.append(s_g)
    return sm


def _extract1(row_v, k, carry, iot, big):
    s0, s1, idx_a, idx_b = carry
    gm = _vmax16(jnp.maximum(s0, s1))
    key = jnp.minimum(jnp.where(s0 == gm, iot, big),
                      jnp.where(s1 == gm, iot + L, big))
    cstar = _vmin16(key)
    cs = cstar[0]
    v0 = row_v[pl.ds(cs * 2 * L, L)]
    v1 = row_v[pl.ds(cs * 2 * L + L, L)]
    lane = _vmin16(jnp.minimum(jnp.where(v0 == gm, iot, big),
                               jnp.where(v1 == gm, iot + L, big)))
    gi = cstar * 2 * L + lane
    v0 = jnp.where(iot == lane, NEG_INF, v0)
    v1 = jnp.where(iot == lane - L, NEG_INF, v1)
    row_v[pl.ds(cs * 2 * L, L)] = v0
    row_v[pl.ds(cs * 2 * L + L, L)] = v1
    nmax = _vmax16(jnp.maximum(v0, v1))
    cl = cs & (L - 1)
    g = cs >> 4
    nolane = np.int32(99)
    s0 = jnp.where(iot == jnp.where(g == 0, cl, nolane), nmax, s0)
    s1 = jnp.where(iot == jnp.where(g == 1, cl, nolane), nmax, s1)
    idx_a = jnp.where(iot == k, gi, idx_a)
    idx_b = jnp.where(iot == k - 8, gi, idx_b)
    return s0, s1, idx_a, idx_b


def _topk2_into(row_a, row_b, tidx_a, tidx_b, iot, big):
    sma = _summary2(row_a, iot)
    smb = _summary2(row_b, iot)

    def extract2(k, carry):
        ca = _extract1(row_a, k, carry[0:4], iot, big)
        cb = _extract1(row_b, k, carry[4:8], iot, big)
        return ca + cb
    carry = lax.fori_loop(0, K, extract2,
                          (sma[0], sma[1], iot, iot,
                           smb[0], smb[1], iot, iot))
    tidx_a[pl.ds(0, L)] = carry[2]
    tidx_a[pl.ds(8, L)] = carry[3]
    tidx_b[pl.ds(0, L)] = carry[6]
    tidx_b[pl.ds(8, L)] = carry[7]


def _sum_into(rows_v, acc_v):
    def percc(cc, _):
        def accum(rr, a):
            return a + rows_v[rr, pl.ds(cc * L, L)]
        acc_v[pl.ds(cc * L, L)] = lax.fori_loop(
            1, K, accum, rows_v[0, pl.ds(cc * L, L)], unroll=4)
        return 0
    lax.fori_loop(0, C // L, percc, 0)


def _sc_body(dist_hbm, xt_hbm, nbr_hbm,
             row_a, row_b, tidx_a, tidx_b, gat_a, gat_b, acc_a, acc_b,
             sem_r, sem_ga, sem_gb, sem_oa, sem_ob):
    wid = lax.axis_index("c") * NS + lax.axis_index("s")
    base = wid * RPW
    iot = _iota16()
    big = iot + np.int32(4096)

    pltpu.async_copy(dist_hbm.at[base], row_a, sem_r)
    pltpu.async_copy(dist_hbm.at[base + 1], row_b, sem_r)

    def step(i, _):
        r0 = base + 2 * i
        r1 = r0 + 1
        r2 = jnp.minimum(r0 + 2, base + RPW - 1)
        r3 = jnp.minimum(r0 + 3, base + RPW - 1)
        pltpu.make_async_copy(dist_hbm.at[r0], row_a, sem_r).wait()
        _topk_into(row_a, tidx_a, iot, big)
        pltpu.async_copy(xt_hbm.at[tidx_a], gat_a, sem_ga)
        pltpu.make_async_copy(dist_hbm.at[r1], row_b, sem_r).wait()
        pltpu.async_copy(dist_hbm.at[r2], row_a, sem_r)
        _topk_into(row_b, tidx_b, iot, big)
        pltpu.async_copy(xt_hbm.at[tidx_b], gat_b, sem_gb)
        pltpu.async_copy(dist_hbm.at[r3], row_b, sem_r)

        pltpu.make_async_copy(xt_hbm.at[tidx_a], gat_a, sem_ga).wait()

        @pl.when(i > 0)
        def _():
            pltpu.make_async_copy(acc_a, nbr_hbm.at[r0], sem_oa).wait()
        _sum_into(gat_a, acc_a)
        pltpu.async_copy(acc_a, nbr_hbm.at[r0], sem_oa)

        pltpu.make_async_copy(xt_hbm.at[tidx_b], gat_b, sem_gb).wait()

        @pl.when(i > 0)
        def _():
            pltpu.make_async_copy(acc_b, nbr_hbm.at[r1], sem_ob).wait()
        _sum_into(gat_b, acc_b)
        pltpu.async_copy(acc_b, nbr_hbm.at[r1], sem_ob)
        return 0

    lax.fori_loop(0, RPW // 2, step, 0)
    last = base + RPW - 1
    pltpu.make_async_copy(dist_hbm.at[last], row_a, sem_r).wait()
    pltpu.make_async_copy(dist_hbm.at[last], row_b, sem_r).wait()
    pltpu.make_async_copy(acc_a, nbr_hbm.at[last], sem_oa).wait()
    pltpu.make_async_copy(acc_b, nbr_hbm.at[last], sem_ob).wait()


def _sc_topk_gather(dist, xt):
    mesh = plsc.VectorSubcoreMesh(core_axis_name="c", subcore_axis_name="s",
                                  num_cores=NC, num_subcores=NS)
    f = pl.kernel(
        _sc_body,
        out_type=jax.ShapeDtypeStruct((N, C), jnp.float32),
        mesh=mesh,
        scratch_types=[
            pltpu.VMEM((N,), jnp.float32),
            pltpu.VMEM((N,), jnp.float32),
            pltpu.VMEM((24,), jnp.int32),
            pltpu.VMEM((24,), jnp.int32),
            pltpu.VMEM((24, C), jnp.float32),
            pltpu.VMEM((24, C), jnp.float32),
            pltpu.VMEM((C,), jnp.float32),
            pltpu.VMEM((C,), jnp.float32),
            pltpu.SemaphoreType.DMA,
            pltpu.SemaphoreType.DMA,
            pltpu.SemaphoreType.DMA,
            pltpu.SemaphoreType.DMA,
            pltpu.SemaphoreType.DMA,
        ],
    )
    return f(dist, xt)


def _final_body(x_ref, eps_ref, nbr_ref, w_ref, o_ref):
    h1 = (1.0 + eps_ref[...]) * x_ref[...]
    w = w_ref[...]
    o_ref[...] = (
        lax.dot_general(w, h1, (((1,), (0,)), ((), ())),
                        preferred_element_type=jnp.float32)
        + lax.dot_general(w, nbr_ref[...], (((1,), (1,)), ((), ())),
                          preferred_element_type=jnp.float32))


def _final(x, eps2, nbr, w2):
    return pl.pallas_call(
        _final_body,
        out_shape=jax.ShapeDtypeStruct((O, N), jnp.float32),
    )(x, eps2, nbr, w2)


def kernel(x, W, eps):
    xt = x.T
    eps2 = eps[:, :, 0]
    w2 = W[:, :, 0, 0]
    dist = _dist(x)
    nbr = _sc_topk_gather(dist, xt)
    return _final(x, eps2, nbr, w2)


# speedup vs baseline: 1.0332x; 1.0332x over previous
import functools

import jax
import jax.numpy as jnp
import numpy as np
from jax import lax
from jax.experimental import pallas as pl
from jax.experimental.pallas import tpu as pltpu
from jax.experimental.pallas import tpu_sc as plsc

N = 1024
C = 256
O = 256
K = 20
NC = 2
NS = 16
L = 16
NW = NC * NS
RPW = N // NW
NCHUNK = N // L
NEG_INF = np.float32(-np.inf)


def _dist_body(x_ref, o_ref):
    x = x_ref[...]
    g = lax.dot_general(x, x, (((0,), (0,)), ((), ())),
                        preferred_element_type=jnp.float32)
    x2 = jnp.sum(x * x, axis=0)
    o_ref[...] = 2.0 * g - x2[None, :] - x2[:, None]


def _dist(x):
    return pl.pallas_call(
        _dist_body,
        out_shape=jax.ShapeDtypeStruct((N, N), jnp.float32),
    )(x)


def _iota16():
    return lax.broadcasted_iota(jnp.int32, (L,), 0)


_GATHER_DNUMS = lax.GatherDimensionNumbers(
    offset_dims=(), collapsed_slice_dims=(0,), start_index_map=(0,))


def _shuffle(v, perm):
    return lax.gather(v, perm[:, None], _GATHER_DNUMS, (1,),
                      mode=lax.GatherScatterMode.PROMISE_IN_BOUNDS)


def _vmax16(v):
    for s in (8, 4, 2, 1):
        v = jnp.maximum(v, _shuffle(v, _iota16() ^ s))
    return v


def _vmin16(v):
    for s in (8, 4, 2, 1):
        v = jnp.minimum(v, _shuffle(v, _iota16() ^ s))
    return v


def _topk_into(row_v, topidx_v, iot, big):
    nolane = np.int32(99)

    def csum(c, carry):
        s0, s1 = carry
        b = c * 2 * L
        cmax = _vmax16(jnp.maximum(row_v[pl.ds(b, L)],
                                   row_v[pl.ds(b + L, L)]))
        cl = c & (L - 1)
        g = c >> 4
        s0 = jnp.where(iot == jnp.where(g == 0, cl, nolane), cmax, s0)
        s1 = jnp.where(iot == jnp.where(g == 1, cl, nolane), cmax, s1)
        return s0, s1
    init = row_v[pl.ds(0, L)]
    sm = lax.fori_loop(0, 2 * L, csum, (init, init))

    def extract(k, carry):
        s0, s1, idx_a, idx_b = carry
        gm = _vmax16(jnp.maximum(s0, s1))
        key = jnp.minimum(jnp.where(s0 == gm, iot, big),
                          jnp.where(s1 == gm, iot + L, big))
        cstar = _vmin16(key)
        cs = cstar[0]
        v0 = row_v[pl.ds(cs * 2 * L, L)]
        v1 = row_v[pl.ds(cs * 2 * L + L, L)]
        lane = _vmin16(jnp.minimum(jnp.where(v0 == gm, iot, big),
                                   jnp.where(v1 == gm, iot + L, big)))
        gi = cstar * 2 * L + lane
        v0 = jnp.where(iot == lane, NEG_INF, v0)
        v1 = jnp.where(iot == lane - L, NEG_INF, v1)
        row_v[pl.ds(cs * 2 * L, L)] = v0
        row_v[pl.ds(cs * 2 * L + L, L)] = v1
        nmax = _vmax16(jnp.maximum(v0, v1))
        cl = cs & (L - 1)
        g = cs >> 4
        nolane = np.int32(99)
        s0 = jnp.where(iot == jnp.where(g == 0, cl, nolane), nmax, s0)
        s1 = jnp.where(iot == jnp.where(g == 1, cl, nolane), nmax, s1)
        idx_a = jnp.where(iot == k, gi, idx_a)
        idx_b = jnp.where(iot == k - 8, gi, idx_b)
        return s0, s1, idx_a, idx_b
    carry = lax.fori_loop(0, K, extract, (sm[0], sm[1], iot, iot))
    topidx_v[pl.ds(0, L)] = carry[2]
    topidx_v[pl.ds(8, L)] = carry[3]


def _summary2(row_v, iot):
    sm = []
    for g in range(2):
        s_g = None
        for l in range(L):
            b = (g * L + l) * 2 * L
            cmax = _vmax16(jnp.maximum(row_v[pl.ds(b, L)],
                                       row_v[pl.ds(b + L, L)]))
            s_g = cmax if s_g is None else jnp.where(iot == l, cmax, s_g)
        sm.append(s_g)
    return sm


def _extract1(row_v, k, carry, iot, big):
    s0, s1, idx_a, idx_b = carry
    gm = _vmax16(jnp.maximum(s0, s1))
    key = jnp.minimum(jnp.where(s0 == gm, iot, big),
                      jnp.where(s1 == gm, iot + L, big))
    cstar = _vmin16(key)
    cs = cstar[0]
    v0 = row_v[pl.ds(cs * 2 * L, L)]
    v1 = row_v[pl.ds(cs * 2 * L + L, L)]
    lane = _vmin16(jnp.minimum(jnp.where(v0 == gm, iot, big),
                               jnp.where(v1 == gm, iot + L, big)))
    gi = cstar * 2 * L + lane
    v0 = jnp.where(iot == lane, NEG_INF, v0)
    v1 = jnp.where(iot == lane - L, NEG_INF, v1)
    row_v[pl.ds(cs * 2 * L, L)] = v0
    row_v[pl.ds(cs * 2 * L + L, L)] = v1
    nmax = _vmax16(jnp.maximum(v0, v1))
    cl = cs & (L - 1)
    g = cs >> 4
    nolane = np.int32(99)
    s0 = jnp.where(iot == jnp.where(g == 0, cl, nolane), nmax, s0)
    s1 = jnp.where(iot == jnp.where(g == 1, cl, nolane), nmax, s1)
    idx_a = jnp.where(iot == k, gi, idx_a)
    idx_b = jnp.where(iot == k - 8, gi, idx_b)
    return s0, s1, idx_a, idx_b


def _topk2_into(row_a, row_b, tidx_a, tidx_b, iot, big):
    sma = _summary2(row_a, iot)
    smb = _summary2(row_b, iot)

    def extract2(k, carry):
        ca = _extract1(row_a, k, carry[0:4], iot, big)
        cb = _extract1(row_b, k, carry[4:8], iot, big)
        return ca + cb
    carry = lax.fori_loop(0, K, extract2,
                          (sma[0], sma[1], iot, iot,
                           smb[0], smb[1], iot, iot))
    tidx_a[pl.ds(0, L)] = carry[2]
    tidx_a[pl.ds(8, L)] = carry[3]
    tidx_b[pl.ds(0, L)] = carry[6]
    tidx_b[pl.ds(8, L)] = carry[7]


def _sum_into(rows_v, acc_v):
    def percc(cc, _):
        def accum(rr, a):
            return a + rows_v[rr, pl.ds(cc * L, L)]
        acc_v[pl.ds(cc * L, L)] = lax.fori_loop(
            1, K, accum, rows_v[0, pl.ds(cc * L, L)], unroll=4)
        return 0
    lax.fori_loop(0, C // L, percc, 0)


def _sc_body(dist_hbm, xt_hbm, nbr_hbm,
             row_a, row_b, tidx_a, tidx_b, gat_a, gat_b, acc_a, acc_b,
             sem_r, sem_ga, sem_gb, sem_oa, sem_ob):
    wid = lax.axis_index("c") * NS + lax.axis_index("s")
    base = wid * RPW
    iot = _iota16()
    big = iot + np.int32(4096)

    pltpu.async_copy(dist_hbm.at[base], row_a, sem_r)
    pltpu.async_copy(dist_hbm.at[base + 1], row_b, sem_r)

    def step(i, _):
        r0 = base + 2 * i
        r1 = r0 + 1
        r2 = jnp.minimum(r0 + 2, base + RPW - 1)
        r3 = jnp.minimum(r0 + 3, base + RPW - 1)
        pltpu.make_async_copy(dist_hbm.at[r0], row_a, sem_r).wait()
        _topk_into(row_a, tidx_a, iot, big)
        pltpu.async_copy(xt_hbm.at[tidx_a], gat_a, sem_ga)
        pltpu.make_async_copy(dist_hbm.at[r1], row_b, sem_r).wait()
        pltpu.async_copy(dist_hbm.at[r2], row_a, sem_r)
        _topk_into(row_b, tidx_b, iot, big)
        pltpu.async_copy(xt_hbm.at[tidx_b], gat_b, sem_gb)
        pltpu.async_copy(dist_hbm.at[r3], row_b, sem_r)

        pltpu.make_async_copy(xt_hbm.at[tidx_a], gat_a, sem_ga).wait()

        @pl.when(i > 0)
        def _():
            pltpu.make_async_copy(acc_a, nbr_hbm.at[r0], sem_oa).wait()
        _sum_into(gat_a, acc_a)
        pltpu.async_copy(acc_a, nbr_hbm.at[r0], sem_oa)

        pltpu.make_async_copy(xt_hbm.at[tidx_b], gat_b, sem_gb).wait()

        @pl.when(i > 0)
        def _():
            pltpu.make_async_copy(acc_b, nbr_hbm.at[r1], sem_ob).wait()
        _sum_into(gat_b, acc_b)
        pltpu.async_copy(acc_b, nbr_hbm.at[r1], sem_ob)
        return 0

    lax.fori_loop(0, RPW // 2, step, 0)
    last = base + RPW - 1
    pltpu.make_async_copy(dist_hbm.at[last], row_a, sem_r).wait()
    pltpu.make_async_copy(dist_hbm.at[last], row_b, sem_r).wait()
    pltpu.make_async_copy(acc_a, nbr_hbm.at[last], sem_oa).wait()
    pltpu.make_async_copy(acc_b, nbr_hbm.at[last], sem_ob).wait()


def _sc_topk_gather(dist, xt):
    mesh = plsc.VectorSubcoreMesh(core_axis_name="c", subcore_axis_name="s",
                                  num_cores=NC, num_subcores=NS)
    f = pl.kernel(
        _sc_body,
        out_type=jax.ShapeDtypeStruct((N, C), jnp.float32),
        mesh=mesh,
        scratch_types=[
            pltpu.VMEM((N,), jnp.float32),
            pltpu.VMEM((N,), jnp.float32),
            pltpu.VMEM((24,), jnp.int32),
            pltpu.VMEM((24,), jnp.int32),
            pltpu.VMEM((24, C), jnp.float32),
            pltpu.VMEM((24, C), jnp.float32),
            pltpu.VMEM((C,), jnp.float32),
            pltpu.VMEM((C,), jnp.float32),
            pltpu.SemaphoreType.DMA,
            pltpu.SemaphoreType.DMA,
            pltpu.SemaphoreType.DMA,
            pltpu.SemaphoreType.DMA,
            pltpu.SemaphoreType.DMA,
        ],
    )
    return f(dist, xt)


def _final_body(xt_ref, epst_ref, nbr_ref, w_ref, o_ref):
    ht = (1.0 + epst_ref[...]) * xt_ref[...] + nbr_ref[...]
    o_ref[...] = lax.dot_general(
        w_ref[...], ht, (((1,), (1,)), ((), ())),
        preferred_element_type=jnp.float32)


def _final(xt, epst, nbr, w2):
    return pl.pallas_call(
        _final_body,
        out_shape=jax.ShapeDtypeStruct((O, N), jnp.float32),
    )(xt, epst, nbr, w2)


def kernel(x, W, eps):
    xt = x.T
    epst = eps[:, :, 0].T
    w2 = W[:, :, 0, 0]
    dist = _dist(x)
    nbr = _sc_topk_gather(dist, xt)
    return _final(xt, epst, nbr, w2)
